# baseline (speedup 1.0000x reference)
import functools

import numpy as np
import jax
import jax.numpy as jnp
from jax import lax
from jax.experimental import pallas as pl
from jax.experimental.pallas import tpu as pltpu
from jax.experimental.pallas import tpu_sc as plsc

NN = 10000
EE = 320000
DD = 128
DE = 16
NPAD = 10240
EPAD = 327680
EPT = 10240
NPT = 640
EPTS = EPAD // 16
CE = 64
NCH = EPTS // CE
CEM = 32
NCHM = EPTS // CEM
TRASH = 10200

_OFF = np.linspace(np.float32(0.0), np.float32(10.0), DE).astype(np.float32)
_COEFF = float(np.float32(-0.5) / np.float32(_OFF[1] - _OFF[0]) ** 2)

_MESH = dict(core_axis_name="c", subcore_axis_name="s",
             num_cores=1)


def _sc_prep(posP, atp1, srcA, dstA, emb):
    @functools.partial(
        pl.kernel,
        out_type=(jax.ShapeDtypeStruct((EPAD, DE), jnp.float32),
                  jax.ShapeDtypeStruct((NPAD, DD), jnp.float32)),
        mesh=plsc.VectorSubcoreMesh(**_MESH),
        scratch_types=[
            pltpu.VMEM((CE,), jnp.int32), pltpu.VMEM((CE,), jnp.int32),
            pltpu.VMEM((CE,), jnp.int32), pltpu.VMEM((CE,), jnp.int32),
            pltpu.VMEM((CE, DD), jnp.float32),
            pltpu.VMEM((CE, DD), jnp.float32),
            pltpu.VMEM((CE, DD), jnp.float32),
            pltpu.VMEM((CE, DD), jnp.float32),
            pltpu.VMEM((CE, DE), jnp.float32),
            pltpu.VMEM((CE, DE), jnp.float32),
            pltpu.VMEM((NPT,), jnp.int32),
            pltpu.VMEM((64, DD), jnp.float32),
            pltpu.SemaphoreType.DMA, pltpu.SemaphoreType.DMA,
            pltpu.SemaphoreType.DMA, pltpu.SemaphoreType.DMA,
            pltpu.SemaphoreType.DMA, pltpu.SemaphoreType.DMA,
        ],
    )
    def k(pos_h, atp_h, src_h, dst_h, emb_h, sq_h, hout_h,
          siA, diA, siB, diB, paA, pbA, paB, pbB, sqA, sqB, ai, hr,
          smiA, smiB, smgA, smgB, smhA, smhB):
        sid = lax.axis_index("s")
        wid = sid
        ebase = sid * EPTS
        nch2 = NCH // 2

        pltpu.sync_copy(atp_h.at[pl.ds(wid * NPT, NPT)], ai)
        for j in range(NPT // 64):
            pltpu.sync_copy(emb_h.at[ai.at[pl.ds(j * 64, 64)]], hr)
            pltpu.sync_copy(
                hr, hout_h.at[pl.ds(pl.multiple_of(wid * NPT + j * 64, 64),
                                    64)])

        def cb(c):
            return pl.ds(pl.multiple_of(ebase + c * CE, CE), CE)

        def issue_idx(c, si, di, sm):
            pltpu.async_copy(src_h.at[cb(c)], si, sm)
            pltpu.async_copy(dst_h.at[cb(c)], di, sm)

        def wait_idx(si, di, sm):
            pltpu.make_async_copy(src_h.at[cb(0)], si, sm).wait()
            pltpu.make_async_copy(dst_h.at[cb(0)], di, sm).wait()

        def issue_g(si, di, pa, pb, sm, sm2):
            pltpu.async_copy(pos_h.at[si], pa, sm)
            pltpu.async_copy(pos_h.at[di], pb, sm2)

        def wait_g(si, di, pa, pb, sm, sm2):
            pltpu.make_async_copy(pos_h.at[si], pa, sm).wait()
            pltpu.make_async_copy(pos_h.at[di], pb, sm2).wait()

        def process(c, pa, pb, sqv):
            @plsc.parallel_loop(0, CE, unroll=8)
            def sqr(e):
                d = pa[e, pl.ds(0, DE)] - pb[e, pl.ds(0, DE)]
                sqv[e, pl.ds(0, DE)] = d * d
            pltpu.sync_copy(sqv, sq_h.at[cb(c)])

        issue_idx(0, siA, diA, smiA)
        issue_idx(1, siB, diB, smiB)
        wait_idx(siA, diA, smiA)
        issue_g(siA, diA, paA, pbA, smgA, smhA)

        def pair(p, carry):
            c0 = 2 * p
            wait_idx(siB, diB, smiB)
            issue_g(siB, diB, paB, pbB, smgB, smhB)
            wait_g(siA, diA, paA, pbA, smgA, smhA)
            process(c0, paA, pbA, sqA)

            @pl.when(p + 1 < nch2)
            def _():
                issue_idx(c0 + 2, siA, diA, smiA)
                wait_idx(siA, diA, smiA)
                issue_g(siA, diA, paA, pbA, smgA, smhA)

            wait_g(siB, diB, paB, pbB, smgB, smhB)
            process(c0 + 1, paB, pbB, sqB)

            @pl.when(p + 1 < nch2)
            def _():
                issue_idx(c0 + 3, siB, diB, smiB)

            return carry

        lax.fori_loop(0, nch2, pair, 0)

    return k(posP, atp1, srcA, dstA, emb)


def _edge_mlp(sq, ew):
    TE = 2048
    full = lambda s: pl.BlockSpec(s, lambda i: (0, 0))
    step = float(_OFF[1])

    def body(sq_ref, w00, b00, g0, t0, w10, b10,
             w01, b01, g1, t1, w11, b11, o0, o1):
        d2 = jnp.sum(sq_ref[...], axis=-1, keepdims=True)
        d = jnp.sqrt(d2)
        offs = lax.broadcasted_iota(
            jnp.int32, (TE, DE), 1).astype(jnp.float32) * step
        t = d - offs
        x = jnp.exp(_COEFF * (t * t))
        for (w0, b0, g, t, w1, b1, o) in (
                (w00, b00, g0, t0, w10, b10, o0),
                (w01, b01, g1, t1, w11, b11, o1)):
            a = jnp.dot(x.astype(jnp.bfloat16), w0[...].astype(jnp.bfloat16),
                        preferred_element_type=jnp.float32)
            a = a + b0[...]
            mu = jnp.mean(a, axis=-1, keepdims=True)
            v = jnp.mean((a - mu) ** 2, axis=-1, keepdims=True)
            a = (a - mu) / jnp.sqrt(v + 1e-5) * g[...] + t[...]
            a = a * jax.nn.sigmoid(a)
            a = jnp.dot(a.astype(jnp.bfloat16), w1[...].astype(jnp.bfloat16),
                        preferred_element_type=jnp.float32)
            o[...] = a + b1[...]

    wspecs = []
    for _ in range(2):
        wspecs += [full((DE, DD)), full((1, DD)), full((1, DD)),
                   full((1, DD)), full((DD, DD)), full((1, DD))]
    return pl.pallas_call(
        body,
        grid=(EPAD // TE,),
        in_specs=[pl.BlockSpec((TE, DE), lambda i: (i, 0))] + wspecs,
        out_specs=[pl.BlockSpec((TE, DD), lambda i: (i, 0))] * 2,
        out_shape=[jax.ShapeDtypeStruct((EPAD, DD), jnp.float32)] * 2,
    )(sq, *ew)


def _msg_scatter(h, ea, srcA, dstC):
    @functools.partial(
        pl.kernel,
        out_type=jax.ShapeDtypeStruct((NPAD, DD), jnp.float32),
        mesh=plsc.VectorSubcoreMesh(**_MESH),
        scratch_types=[
            pltpu.VMEM((CEM,), jnp.int32), pltpu.VMEM((CEM,), jnp.int32),
            pltpu.VMEM((CEM,), jnp.int32), pltpu.VMEM((CEM,), jnp.int32),
            pltpu.VMEM((CEM, DD), jnp.float32),
            pltpu.VMEM((CEM, DD), jnp.float32),
            pltpu.VMEM((CEM, DD), jnp.float32),
            pltpu.VMEM((CEM, DD), jnp.float32),
            pltpu.VMEM_SHARED((NPAD, DD), jnp.float32),
            pltpu.SemaphoreType.DMA, pltpu.SemaphoreType.DMA,
            pltpu.SemaphoreType.DMA, pltpu.SemaphoreType.DMA,
            pltpu.SemaphoreType.DMA, pltpu.SemaphoreType.DMA,
        ],
    )
    def k(h_h, ea_h, src_h, dst_h, out_h,
          siA, diA, siB, diB, hbA, eaA, hbB, eaB, acc,
          smiA, smiB, smgA, smgB, smeA, smeB):
        sid = lax.axis_index("s")
        ebase = sid * EPTS
        nch2 = NCHM // 2
        row0 = pl.multiple_of(sid * (NPAD // 16), NPAD // 16)

        @plsc.parallel_loop(0, CEM, unroll=4)
        def zb(i):
            for r in range(8):
                eaA[i, pl.ds(r * 16, 16)] = jnp.zeros((16,), jnp.float32)
        for j in range(NPAD // 16 // CEM):
            pltpu.sync_copy(eaA, acc.at[pl.ds(row0 + j * CEM, CEM)])
        plsc.subcore_barrier()

        def cb(c):
            return pl.ds(pl.multiple_of(ebase + c * CEM, CEM), CEM)

        def issue_idx(c, si, di, sm):
            pltpu.async_copy(src_h.at[cb(c)], si, sm)
            pltpu.async_copy(dst_h.at[cb(c)], di, sm)

        def wait_idx(si, di, sm):
            pltpu.make_async_copy(src_h.at[cb(0)], si, sm).wait()
            pltpu.make_async_copy(dst_h.at[cb(0)], di, sm).wait()

        def issue_g(c, si, hb, eab, smg, sme):
            pltpu.async_copy(h_h.at[si], hb, smg)
            pltpu.async_copy(ea_h.at[cb(c)], eab, sme)

        def wait_g(si, hb, eab, smg, sme):
            pltpu.make_async_copy(h_h.at[si], hb, smg).wait()
            pltpu.make_async_copy(ea_h.at[cb(0)], eab, sme).wait()

        def process(hb, eab, di):
            @plsc.parallel_loop(0, CEM, unroll=4)
            def mul(e):
                for r in range(8):
                    s = pl.ds(r * 16, 16)
                    eab[e, s] = eab[e, s] * hb[e, s]
            pltpu.sync_copy(eab, acc.at[di], add=True)

        issue_idx(0, siA, diA, smiA)
        issue_idx(1, siB, diB, smiB)
        wait_idx(siA, diA, smiA)
        issue_g(0, siA, hbA, eaA, smgA, smeA)

        def pair(p, carry):
            c0 = 2 * p
            wait_idx(siB, diB, smiB)
            issue_g(c0 + 1, siB, hbB, eaB, smgB, smeB)
            wait_g(siA, hbA, eaA, smgA, smeA)
            process(hbA, eaA, diA)

            @pl.when(p + 1 < nch2)
            def _():
                issue_idx(c0 + 2, siA, diA, smiA)
                wait_idx(siA, diA, smiA)
                issue_g(c0 + 2, siA, hbA, eaA, smgA, smeA)

            wait_g(siB, hbB, eaB, smgB, smeB)
            process(hbB, eaB, diB)

            @pl.when(p + 1 < nch2)
            def _():
                issue_idx(c0 + 3, siB, diB, smiB)

            return carry

        lax.fori_loop(0, nch2, pair, 0)
        plsc.subcore_barrier()
        pltpu.sync_copy(acc.at[pl.ds(row0, NPAD // 16)],
                        out_h.at[pl.ds(row0, NPAD // 16)])

    return k(h, ea, srcA, dstC)


def _node_mlp(hp0, h, nw, first):
    TR = 1024
    full = lambda s: pl.BlockSpec(s, lambda i: (0, 0))
    row = pl.BlockSpec((TR, DD), lambda i: (i, 0))

    def body(hp0_r, h_r, ca_r, w0_r, b0_r, g_r, t_r, w1_r, b1_r,
             fg_r, fb_r, z_r, s_r):
        pi = pl.program_id(0)
        zin = ca_r[...] * hp0_r[...] + h_r[...]
        a = jnp.dot(zin, w0_r[...], preferred_element_type=jnp.float32)
        a = a + b0_r[...]
        mu = jnp.mean(a, axis=-1, keepdims=True)
        v = jnp.mean((a - mu) ** 2, axis=-1, keepdims=True)
        a = (a - mu) / jnp.sqrt(v + 1e-5) * g_r[...] + t_r[...]
        a = a * jax.nn.sigmoid(a)
        z = jnp.dot(a, w1_r[...], preferred_element_type=jnp.float32)
        z = z + b1_r[...]
        if first:
            mu2 = jnp.mean(z, axis=-1, keepdims=True)
            v2 = jnp.mean((z - mu2) ** 2, axis=-1, keepdims=True)
            z = (z - mu2) / jnp.sqrt(v2 + 1e-5) * fg_r[...] + fb_r[...]
            z = z * jax.nn.sigmoid(z)
        z_r[...] = z
        rows = pi * TR + lax.broadcasted_iota(jnp.int32, (TR, DD), 0)
        zm = jnp.where(rows < NN, z, 0.0)
        part = jnp.concatenate(
            [jnp.sum(zm, axis=0, keepdims=True),
             jnp.sum(zm * zm, axis=0, keepdims=True),
             jnp.zeros((6, DD), jnp.float32)], axis=0)

        @pl.when(pi == 0)
        def _():
            s_r[...] = part

        @pl.when(pi != 0)
        def _():
            s_r[...] = s_r[...] + part

    return pl.pallas_call(
        body,
        grid=(NPAD // TR,),
        in_specs=[row, row, full((1, DD)), full((DD, DD)),
                  full((1, DD)), full((1, DD)), full((1, DD)),
                  full((DD, DD)), full((1, DD)), full((1, DD)),
                  full((1, DD))],
        out_specs=[row, pl.BlockSpec((8, DD), lambda i: (0, 0))],
        out_shape=[jax.ShapeDtypeStruct((NPAD, DD), jnp.float32),
                   jax.ShapeDtypeStruct((8, DD), jnp.float32)],
    )(hp0, h, *nw)


def _gnorm_resid(z, sums, h, gw, gb, gms, ga, first):
    TR = 1024
    full = lambda s: pl.BlockSpec(s, lambda i: (0, 0))
    row = pl.BlockSpec((TR, DD), lambda i: (i, 0))

    def body(z_r, s_r, h_r, gw_r, gb_r, gms_r, ga_r, o_r):
        s = s_r[...]
        mean = s[0:1, :] * (1.0 / NN)
        m2 = s[1:2, :] * (1.0 / NN)
        mm = mean * gms_r[...]
        var = m2 - 2.0 * mm * mean + mm * mm
        inv = 1.0 / jnp.sqrt(var + 1e-5)
        zf = (z_r[...] - mm) * (gw_r[...] * inv) + gb_r[...]
        if first:
            zf = zf * jax.nn.sigmoid(zf)
        o_r[...] = ga_r[...] * zf + h_r[...]

    return pl.pallas_call(
        body,
        grid=(NPAD // TR,),
        in_specs=[row, pl.BlockSpec((8, DD), lambda i: (0, 0)), row,
                  full((1, DD)), full((1, DD)), full((1, DD)),
                  full((1, DD))],
        out_specs=row,
        out_shape=jax.ShapeDtypeStruct((NPAD, DD), jnp.float32),
    )(z, sums, h, gw, gb, gms, ga)


def kernel(atom_type, pos, batch, edge_index, params):
    del batch
    p = params
    r1 = lambda a: a.reshape(1, DD)
    src = edge_index[0]
    dst = edge_index[1]
    pe = EPAD - EE
    srcA = jnp.concatenate([src, jnp.zeros((pe,), jnp.int32)])
    dstA = jnp.concatenate([dst, jnp.zeros((pe,), jnp.int32)])
    dstC = jnp.concatenate([dst, jnp.full((pe,), TRASH, jnp.int32)])
    atp1 = jnp.concatenate(
        [atom_type, jnp.zeros((NPAD - NN,), jnp.int32)])
    posP = jnp.pad(pos, ((0, 0), (0, DD - 3)))

    sq, h = _sc_prep(posP, atp1, srcA, dstA, p['emb'])

    ew = []
    for i in range(2):
        ew += [p['eW0_%d' % i], r1(p['eb0_%d' % i]), r1(p['eln_g_%d' % i]),
               r1(p['eln_b_%d' % i]), p['eW1_%d' % i], r1(p['eb1_%d' % i])]
    ea = _edge_mlp(sq, ew)

    for i in range(2):
        first = i == 0
        h0 = _msg_scatter(h, ea[i], srcA, dstC)
        nw = [r1(p['ca_%d' % i]), p['nW0_%d' % i], r1(p['nb0_%d' % i]),
              r1(p['nln_g_%d' % i]), r1(p['nln_b_%d' % i]),
              p['nW1_%d' % i], r1(p['nb1_%d' % i]),
              r1(p['nfln_g_0']), r1(p['nfln_b_0'])]
        z, sums = _node_mlp(h0, h, nw, first)
        ga = jnp.full((1, DD), 1.0, jnp.float32) * p['galpha'][i]
        h = _gnorm_resid(z, sums, h, r1(p['gn_w_%d' % i]),
                         r1(p['gn_b_%d' % i]), r1(p['gn_ms_%d' % i]),
                         ga, first)
    return h[:NN]


# speedup vs baseline: 1.2551x; 1.2551x over previous
import functools

import numpy as np
import jax
import jax.numpy as jnp
from jax import lax
from jax.experimental import pallas as pl
from jax.experimental.pallas import tpu as pltpu
from jax.experimental.pallas import tpu_sc as plsc

NN = 10000
EE = 320000
DD = 128
DE = 16
NPAD = 10240
EPAD = 327680
EPT = 10240
NPT = 320
EPTS = EPAD // 16
CE = 64
NCH0 = 224
NCH1 = 96
TRASH = 10200

_OFF = np.linspace(np.float32(0.0), np.float32(10.0), DE).astype(np.float32)
_COEFF = float(np.float32(-0.5) / np.float32(_OFF[1] - _OFF[0]) ** 2)

_MESH = dict(core_axis_name="c", subcore_axis_name="s")


def _sc_prep(posP, atp1, srcA, dstA, emb):
    @functools.partial(
        pl.kernel,
        out_type=(jax.ShapeDtypeStruct((EPAD, DE), jnp.float32),
                  jax.ShapeDtypeStruct((NPAD, DD), jnp.float32)),
        mesh=plsc.VectorSubcoreMesh(**_MESH),
        scratch_types=[
            pltpu.VMEM((CE,), jnp.int32), pltpu.VMEM((CE,), jnp.int32),
            pltpu.VMEM((CE,), jnp.int32), pltpu.VMEM((CE,), jnp.int32),
            pltpu.VMEM((CE, DD), jnp.float32),
            pltpu.VMEM((CE, DD), jnp.float32),
            pltpu.VMEM((CE, DD), jnp.float32),
            pltpu.VMEM((CE, DD), jnp.float32),
            pltpu.VMEM((CE, DE), jnp.float32),
            pltpu.VMEM((CE, DE), jnp.float32),
            pltpu.VMEM((NPT,), jnp.int32),
            pltpu.VMEM((64, DD), jnp.float32),
            pltpu.SemaphoreType.DMA, pltpu.SemaphoreType.DMA,
            pltpu.SemaphoreType.DMA, pltpu.SemaphoreType.DMA,
            pltpu.SemaphoreType.DMA, pltpu.SemaphoreType.DMA,
        ],
    )
    def k(pos_h, atp_h, src_h, dst_h, emb_h, sq_h, hout_h,
          siA, diA, siB, diB, paA, pbA, paB, pbB, sqA, sqB, ai, hr,
          smiA, smiB, smgA, smgB, smhA, smhB):
        cid = lax.axis_index("c")
        sid = lax.axis_index("s")
        wid = sid * 2 + cid
        ebase = jnp.where(cid == 0, sid * (NCH0 * CE),
                          16 * (NCH0 * CE) + sid * (NCH1 * CE))
        nch2 = jnp.where(cid == 0, NCH0 // 2, NCH1 // 2)

        pltpu.sync_copy(atp_h.at[pl.ds(wid * NPT, NPT)], ai)
        for j in range(NPT // 64):
            pltpu.sync_copy(emb_h.at[ai.at[pl.ds(j * 64, 64)]], hr)
            pltpu.sync_copy(
                hr, hout_h.at[pl.ds(pl.multiple_of(wid * NPT + j * 64, 64),
                                    64)])

        def cb(c):
            return pl.ds(pl.multiple_of(ebase + c * CE, CE), CE)

        def issue_idx(c, si, di, sm):
            pltpu.async_copy(src_h.at[cb(c)], si, sm)
            pltpu.async_copy(dst_h.at[cb(c)], di, sm)

        def wait_idx(si, di, sm):
            pltpu.make_async_copy(src_h.at[cb(0)], si, sm).wait()
            pltpu.make_async_copy(dst_h.at[cb(0)], di, sm).wait()

        def issue_g(si, di, pa, pb, sm, sm2):
            pltpu.async_copy(pos_h.at[si], pa, sm)
            pltpu.async_copy(pos_h.at[di], pb, sm2)

        def wait_g(si, di, pa, pb, sm, sm2):
            pltpu.make_async_copy(pos_h.at[si], pa, sm).wait()
            pltpu.make_async_copy(pos_h.at[di], pb, sm2).wait()

        def process(c, pa, pb, sqv):
            @plsc.parallel_loop(0, CE, unroll=8)
            def sqr(e):
                d = pa[e, pl.ds(0, DE)] - pb[e, pl.ds(0, DE)]
                sqv[e, pl.ds(0, DE)] = d * d
            pltpu.sync_copy(sqv, sq_h.at[cb(c)])

        issue_idx(0, siA, diA, smiA)
        issue_idx(1, siB, diB, smiB)
        wait_idx(siA, diA, smiA)
        issue_g(siA, diA, paA, pbA, smgA, smhA)

        def pair(p, carry):
            c0 = 2 * p
            wait_idx(siB, diB, smiB)
            issue_g(siB, diB, paB, pbB, smgB, smhB)
            wait_g(siA, diA, paA, pbA, smgA, smhA)
            process(c0, paA, pbA, sqA)

            @pl.when(p + 1 < nch2)
            def _():
                issue_idx(c0 + 2, siA, diA, smiA)
                wait_idx(siA, diA, smiA)
                issue_g(siA, diA, paA, pbA, smgA, smhA)

            wait_g(siB, diB, paB, pbB, smgB, smhB)
            process(c0 + 1, paB, pbB, sqB)

            @pl.when(p + 1 < nch2)
            def _():
                issue_idx(c0 + 3, siB, diB, smiB)

            return carry

        lax.fori_loop(0, nch2, pair, 0)

    return k(posP, atp1, srcA, dstA, emb)


def _edge_mlp(sq, ew):
    TE = 2048
    full = lambda s: pl.BlockSpec(s, lambda i: (0, 0))
    step = float(_OFF[1])

    def body(sq_ref, w00, b00, g0, t0, w10, b10,
             w01, b01, g1, t1, w11, b11, o0, o1):
        d2 = jnp.sum(sq_ref[...], axis=-1, keepdims=True)
        d = jnp.sqrt(d2)
        offs = lax.broadcasted_iota(
            jnp.int32, (TE, DE), 1).astype(jnp.float32) * step
        t = d - offs
        x = jnp.exp(_COEFF * (t * t))
        for (w0, b0, g, t, w1, b1, o) in (
                (w00, b00, g0, t0, w10, b10, o0),
                (w01, b01, g1, t1, w11, b11, o1)):
            a = jnp.dot(x.astype(jnp.bfloat16), w0[...].astype(jnp.bfloat16),
                        preferred_element_type=jnp.float32)
            a = a + b0[...]
            mu = jnp.mean(a, axis=-1, keepdims=True)
            v = jnp.mean((a - mu) ** 2, axis=-1, keepdims=True)
            a = (a - mu) / jnp.sqrt(v + 1e-5) * g[...] + t[...]
            a = a * jax.nn.sigmoid(a)
            a = jnp.dot(a.astype(jnp.bfloat16), w1[...].astype(jnp.bfloat16),
                        preferred_element_type=jnp.float32)
            o[...] = a + b1[...]

    wspecs = []
    for _ in range(2):
        wspecs += [full((DE, DD)), full((1, DD)), full((1, DD)),
                   full((1, DD)), full((DD, DD)), full((1, DD))]
    return pl.pallas_call(
        body,
        grid=(EPAD // TE,),
        in_specs=[pl.BlockSpec((TE, DE), lambda i: (i, 0))] + wspecs,
        out_specs=[pl.BlockSpec((TE, DD), lambda i: (i, 0))] * 2,
        out_shape=[jax.ShapeDtypeStruct((EPAD, DD), jnp.float32)] * 2,
    )(sq, *ew)


def _msg_scatter(h, ea, srcA, dstC):
    @functools.partial(
        pl.kernel,
        out_type=jax.ShapeDtypeStruct((2, NPAD, DD), jnp.float32),
        mesh=plsc.VectorSubcoreMesh(**_MESH),
        scratch_types=[
            pltpu.VMEM((CE,), jnp.int32), pltpu.VMEM((CE,), jnp.int32),
            pltpu.VMEM((CE,), jnp.int32), pltpu.VMEM((CE,), jnp.int32),
            pltpu.VMEM((CE, DD), jnp.float32),
            pltpu.VMEM((CE, DD), jnp.float32),
            pltpu.VMEM((CE, DD), jnp.float32),
            pltpu.VMEM((CE, DD), jnp.float32),
            pltpu.VMEM_SHARED((NPAD, DD), jnp.float32),
            pltpu.SemaphoreType.DMA, pltpu.SemaphoreType.DMA,
            pltpu.SemaphoreType.DMA, pltpu.SemaphoreType.DMA,
            pltpu.SemaphoreType.DMA, pltpu.SemaphoreType.DMA,
        ],
    )
    def k(h_h, ea_h, src_h, dst_h, out_h,
          siA, diA, siB, diB, hbA, eaA, hbB, eaB, acc,
          smiA, smiB, smgA, smgB, smeA, smeB):
        cid = lax.axis_index("c")
        sid = lax.axis_index("s")
        ebase = jnp.where(cid == 0, sid * (NCH0 * CE),
                          16 * (NCH0 * CE) + sid * (NCH1 * CE))
        nch2 = jnp.where(cid == 0, NCH0 // 2, NCH1 // 2)
        row0 = pl.multiple_of(sid * (NPAD // 16), NPAD // 16)

        @plsc.parallel_loop(0, CE, unroll=4)
        def zb(i):
            for r in range(8):
                eaA[i, pl.ds(r * 16, 16)] = jnp.zeros((16,), jnp.float32)
        for j in range(NPAD // 16 // CE):
            pltpu.sync_copy(eaA, acc.at[pl.ds(row0 + j * CE, CE)])
        plsc.subcore_barrier()

        def cb(c):
            return pl.ds(pl.multiple_of(ebase + c * CE, CE), CE)

        def issue_idx(c, si, di, sm):
            pltpu.async_copy(src_h.at[cb(c)], si, sm)
            pltpu.async_copy(dst_h.at[cb(c)], di, sm)

        def wait_idx(si, di, sm):
            pltpu.make_async_copy(src_h.at[cb(0)], si, sm).wait()
            pltpu.make_async_copy(dst_h.at[cb(0)], di, sm).wait()

        def issue_g(c, si, hb, eab, smg, sme):
            pltpu.async_copy(h_h.at[si], hb, smg)
            pltpu.async_copy(ea_h.at[cb(c)], eab, sme)

        def wait_g(si, hb, eab, smg, sme):
            pltpu.make_async_copy(h_h.at[si], hb, smg).wait()
            pltpu.make_async_copy(ea_h.at[cb(0)], eab, sme).wait()

        def process(hb, eab, di):
            @plsc.parallel_loop(0, CE, unroll=4)
            def mul(e):
                for r in range(8):
                    s = pl.ds(r * 16, 16)
                    eab[e, s] = eab[e, s] * hb[e, s]
            pltpu.sync_copy(eab, acc.at[di], add=True)

        issue_idx(0, siA, diA, smiA)
        issue_idx(1, siB, diB, smiB)
        wait_idx(siA, diA, smiA)
        issue_g(0, siA, hbA, eaA, smgA, smeA)

        def pair(p, carry):
            c0 = 2 * p
            wait_idx(siB, diB, smiB)
            issue_g(c0 + 1, siB, hbB, eaB, smgB, smeB)
            wait_g(siA, hbA, eaA, smgA, smeA)
            process(hbA, eaA, diA)

            @pl.when(p + 1 < nch2)
            def _():
                issue_idx(c0 + 2, siA, diA, smiA)
                wait_idx(siA, diA, smiA)
                issue_g(c0 + 2, siA, hbA, eaA, smgA, smeA)

            wait_g(siB, hbB, eaB, smgB, smeB)
            process(hbB, eaB, diB)

            @pl.when(p + 1 < nch2)
            def _():
                issue_idx(c0 + 3, siB, diB, smiB)

            return carry

        lax.fori_loop(0, nch2, pair, 0)
        plsc.subcore_barrier()
        pltpu.sync_copy(acc.at[pl.ds(row0, NPAD // 16)],
                        out_h.at[cid].at[pl.ds(row0, NPAD // 16)])

    return k(h, ea, srcA, dstC)


def _node_mlp(hp0, hp1, h, nw, first):
    TR = 1024
    full = lambda s: pl.BlockSpec(s, lambda i: (0, 0))
    row = pl.BlockSpec((TR, DD), lambda i: (i, 0))

    def body(hp0_r, hp1_r, h_r, ca_r, w0_r, b0_r, g_r, t_r, w1_r, b1_r,
             fg_r, fb_r, z_r, s_r):
        pi = pl.program_id(0)
        zin = ca_r[...] * (hp0_r[...] + hp1_r[...]) + h_r[...]
        a = jnp.dot(zin, w0_r[...], preferred_element_type=jnp.float32)
        a = a + b0_r[...]
        mu = jnp.mean(a, axis=-1, keepdims=True)
        v = jnp.mean((a - mu) ** 2, axis=-1, keepdims=True)
        a = (a - mu) / jnp.sqrt(v + 1e-5) * g_r[...] + t_r[...]
        a = a * jax.nn.sigmoid(a)
        z = jnp.dot(a, w1_r[...], preferred_element_type=jnp.float32)
        z = z + b1_r[...]
        if first:
            mu2 = jnp.mean(z, axis=-1, keepdims=True)
            v2 = jnp.mean((z - mu2) ** 2, axis=-1, keepdims=True)
            z = (z - mu2) / jnp.sqrt(v2 + 1e-5) * fg_r[...] + fb_r[...]
            z = z * jax.nn.sigmoid(z)
        z_r[...] = z
        rows = pi * TR + lax.broadcasted_iota(jnp.int32, (TR, DD), 0)
        zm = jnp.where(rows < NN, z, 0.0)
        part = jnp.concatenate(
            [jnp.sum(zm, axis=0, keepdims=True),
             jnp.sum(zm * zm, axis=0, keepdims=True),
             jnp.zeros((6, DD), jnp.float32)], axis=0)

        @pl.when(pi == 0)
        def _():
            s_r[...] = part

        @pl.when(pi != 0)
        def _():
            s_r[...] = s_r[...] + part

    return pl.pallas_call(
        body,
        grid=(NPAD // TR,),
        in_specs=[row, row, row, full((1, DD)), full((DD, DD)),
                  full((1, DD)), full((1, DD)), full((1, DD)),
                  full((DD, DD)), full((1, DD)), full((1, DD)),
                  full((1, DD))],
        out_specs=[row, pl.BlockSpec((8, DD), lambda i: (0, 0))],
        out_shape=[jax.ShapeDtypeStruct((NPAD, DD), jnp.float32),
                   jax.ShapeDtypeStruct((8, DD), jnp.float32)],
    )(hp0, hp1, h, *nw)


def _gnorm_resid(z, sums, h, gw, gb, gms, ga, first):
    TR = 1024
    full = lambda s: pl.BlockSpec(s, lambda i: (0, 0))
    row = pl.BlockSpec((TR, DD), lambda i: (i, 0))

    def body(z_r, s_r, h_r, gw_r, gb_r, gms_r, ga_r, o_r):
        s = s_r[...]
        mean = s[0:1, :] * (1.0 / NN)
        m2 = s[1:2, :] * (1.0 / NN)
        mm = mean * gms_r[...]
        var = m2 - 2.0 * mm * mean + mm * mm
        inv = 1.0 / jnp.sqrt(var + 1e-5)
        zf = (z_r[...] - mm) * (gw_r[...] * inv) + gb_r[...]
        if first:
            zf = zf * jax.nn.sigmoid(zf)
        o_r[...] = ga_r[...] * zf + h_r[...]

    return pl.pallas_call(
        body,
        grid=(NPAD // TR,),
        in_specs=[row, pl.BlockSpec((8, DD), lambda i: (0, 0)), row,
                  full((1, DD)), full((1, DD)), full((1, DD)),
                  full((1, DD))],
        out_specs=row,
        out_shape=jax.ShapeDtypeStruct((NPAD, DD), jnp.float32),
    )(z, sums, h, gw, gb, gms, ga)


def kernel(atom_type, pos, batch, edge_index, params):
    del batch
    p = params
    r1 = lambda a: a.reshape(1, DD)
    src = edge_index[0]
    dst = edge_index[1]
    pe = EPAD - EE
    srcA = jnp.concatenate([src, jnp.zeros((pe,), jnp.int32)])
    dstA = jnp.concatenate([dst, jnp.zeros((pe,), jnp.int32)])
    dstC = jnp.concatenate([dst, jnp.full((pe,), TRASH, jnp.int32)])
    atp1 = jnp.concatenate(
        [atom_type, jnp.zeros((NPAD - NN,), jnp.int32)])
    posP = jnp.pad(pos, ((0, 0), (0, DD - 3)))

    sq, h = _sc_prep(posP, atp1, srcA, dstA, p['emb'])

    ew = []
    for i in range(2):
        ew += [p['eW0_%d' % i], r1(p['eb0_%d' % i]), r1(p['eln_g_%d' % i]),
               r1(p['eln_b_%d' % i]), p['eW1_%d' % i], r1(p['eb1_%d' % i])]
    ea = _edge_mlp(sq, ew)

    for i in range(2):
        first = i == 0
        hp = _msg_scatter(h, ea[i], srcA, dstC)
        nw = [r1(p['ca_%d' % i]), p['nW0_%d' % i], r1(p['nb0_%d' % i]),
              r1(p['nln_g_%d' % i]), r1(p['nln_b_%d' % i]),
              p['nW1_%d' % i], r1(p['nb1_%d' % i]),
              r1(p['nfln_g_0']), r1(p['nfln_b_0'])]
        z, sums = _node_mlp(hp[0], hp[1], h, nw, first)
        ga = jnp.full((1, DD), 1.0, jnp.float32) * p['galpha'][i]
        h = _gnorm_resid(z, sums, h, r1(p['gn_w_%d' % i]),
                         r1(p['gn_b_%d' % i]), r1(p['gn_ms_%d' % i]),
                         ga, first)
    return h[:NN]
